# baseline (speedup 1.0000x reference)
import functools

import jax
import jax.numpy as jnp
from jax import lax
from jax.experimental import pallas as pl
from jax.experimental.pallas import tpu as pltpu
from jax.experimental.pallas import tpu_sc as plsc

_E, _T, _N = 16, 32, 1400
_P = 8
_DE = 16
_QD = 34
_NC, _NS = 2, 16
_NW = _NC * _NS
_BPW = 704
_BPAD = _NW * _BPW


def _sc_gather(table, idx_flat):
    mesh = plsc.VectorSubcoreMesh(core_axis_name="c", subcore_axis_name="s")

    @functools.partial(
        pl.kernel,
        mesh=mesh,
        out_type=jax.ShapeDtypeStruct((_BPAD, _DE), jnp.float32),
        scratch_types=[
            pltpu.VMEM((_BPW,), jnp.int32),
            pltpu.VMEM((_BPW, _DE), jnp.float32),
            pltpu.SemaphoreType.DMA,
        ],
        compiler_params=pltpu.CompilerParams(use_tc_tiling_on_sc=False),
    )
    def gather_kernel(table_hbm, idx_hbm, out_hbm, idx_v, rows_v, sem):
        wid = lax.axis_index("s") * _NC + lax.axis_index("c")
        base = wid * _BPW
        pltpu.sync_copy(idx_hbm.at[pl.ds(base, _BPW)], idx_v)
        pltpu.async_copy(table_hbm.at[idx_v], rows_v, sem).wait()
        pltpu.sync_copy(rows_v, out_hbm.at[pl.ds(base, _BPW)])

    return gather_kernel(table, idx_flat)


_TB = 16


def _assemble_body(yf_ref, xf_ref, emb_ref, sal_ref, sel_ref, pad_ref,
                   out_ref):
    for c in range(_QD):
        if c < 16:
            ref = yf_ref if c < 8 else xf_ref
            src = jnp.nan_to_num(ref[0, c % 8, :].reshape(1, _N))
            data = jnp.broadcast_to(src, (_TB, _N))
        elif c < 32:
            src = jnp.nan_to_num(emb_ref[0, c - 16, :].reshape(1, _N))
            data = jnp.broadcast_to(src, (_TB, _N))
        else:
            ref = sal_ref if c == 32 else sel_ref
            src = jnp.nan_to_num(ref[0, :, :])
            data = jnp.broadcast_to(src, (_TB, _N))
        padv = jnp.broadcast_to(pad_ref[c, :].reshape(1, _P), (_TB, _P))
        out_ref[c, :, :] = jnp.concatenate([padv, data], axis=1)


def kernel(pv, pv_y_osgb_fourier, pv_x_osgb_fourier, pv_system_row_number,
           pv_x_osgb, solar_azimuth, solar_elevation, query_padding,
           embedding_table):
    idx = pv_system_row_number.reshape(-1)
    idx = jnp.concatenate(
        [idx, jnp.zeros((_BPAD - _E * _N,), jnp.int32)])
    emb_flat = _sc_gather(embedding_table, idx)

    yf_t = pv_y_osgb_fourier.transpose(0, 2, 1)
    xf_t = pv_x_osgb_fourier.transpose(0, 2, 1)
    emb_t = emb_flat[:_E * _N].reshape(_E, _N, _DE).transpose(0, 2, 1)
    sal = solar_azimuth.reshape(_E, _T, 1)
    sel = solar_elevation.reshape(_E, _T, 1)
    pad_t = query_padding.T

    n_tb = _T // _TB
    out = pl.pallas_call(
        _assemble_body,
        grid=(_E, n_tb),
        in_specs=[
            pl.BlockSpec((1, _P, _N), lambda e, tb: (e, 0, 0)),
            pl.BlockSpec((1, _P, _N), lambda e, tb: (e, 0, 0)),
            pl.BlockSpec((1, _DE, _N), lambda e, tb: (e, 0, 0)),
            pl.BlockSpec((1, _TB, 1), lambda e, tb: (e, tb, 0)),
            pl.BlockSpec((1, _TB, 1), lambda e, tb: (e, tb, 0)),
            pl.BlockSpec((_QD, _P), lambda e, tb: (0, 0)),
        ],
        out_specs=pl.BlockSpec((_QD, _TB, _P + _N),
                               lambda e, tb: (0, e * (_T // _TB) + tb, 0)),
        out_shape=jax.ShapeDtypeStruct((_QD, _E * _T, _P + _N), jnp.float32),
        compiler_params=pltpu.CompilerParams(
            dimension_semantics=("parallel", "parallel")),
    )(yf_t, xf_t, emb_t, sal, sel, pad_t)
    return jnp.transpose(out, (1, 2, 0))


# speedup vs baseline: 1.1065x; 1.1065x over previous
import functools

import jax
import jax.numpy as jnp
from jax import lax
from jax.experimental import pallas as pl
from jax.experimental.pallas import tpu as pltpu
from jax.experimental.pallas import tpu_sc as plsc

_E, _T, _N = 16, 32, 1400
_P = 8
_DE = 16
_QD = 34
_NC, _NS = 2, 16
_NW = _NC * _NS
_BPW = 704
_BPAD = _NW * _BPW


def _sc_gather(table, idx_flat):
    mesh = plsc.VectorSubcoreMesh(core_axis_name="c", subcore_axis_name="s")

    @functools.partial(
        pl.kernel,
        mesh=mesh,
        out_type=jax.ShapeDtypeStruct((_BPAD, _DE), jnp.float32),
        scratch_types=[
            pltpu.VMEM((_BPW,), jnp.int32),
            pltpu.VMEM((_BPW, _DE), jnp.float32),
            pltpu.SemaphoreType.DMA,
        ],
        compiler_params=pltpu.CompilerParams(use_tc_tiling_on_sc=False),
    )
    def gather_kernel(table_hbm, idx_hbm, out_hbm, idx_v, rows_v, sem):
        wid = lax.axis_index("s") * _NC + lax.axis_index("c")
        base = wid * _BPW
        pltpu.sync_copy(idx_hbm.at[pl.ds(base, _BPW)], idx_v)
        pltpu.async_copy(table_hbm.at[idx_v], rows_v, sem).wait()
        pltpu.sync_copy(rows_v, out_hbm.at[pl.ds(base, _BPW)])

    return gather_kernel(table, idx_flat)


_EB = 2


def _assemble_body(yf_ref, xf_ref, emb_ref, sal_ref, sel_ref, pad_ref,
                   out_ref):
    for c in range(_QD):
        for ex in range(_EB):
            if c < 16:
                ref = yf_ref if c < 8 else xf_ref
                src = jnp.nan_to_num(ref[ex, c % 8, :].reshape(1, _N))
                data = jnp.broadcast_to(src, (_T, _N))
            elif c < 32:
                src = jnp.nan_to_num(emb_ref[ex, c - 16, :].reshape(1, _N))
                data = jnp.broadcast_to(src, (_T, _N))
            else:
                ref = sal_ref if c == 32 else sel_ref
                src = jnp.nan_to_num(ref[ex, :, :])
                data = jnp.broadcast_to(src, (_T, _N))
            padv = jnp.broadcast_to(pad_ref[c, :].reshape(1, _P), (_T, _P))
            out_ref[c, ex * _T:(ex + 1) * _T, :] = jnp.concatenate(
                [padv, data], axis=1)


def kernel(pv, pv_y_osgb_fourier, pv_x_osgb_fourier, pv_system_row_number,
           pv_x_osgb, solar_azimuth, solar_elevation, query_padding,
           embedding_table):
    idx = pv_system_row_number.reshape(-1)
    idx = jnp.concatenate(
        [idx, jnp.zeros((_BPAD - _E * _N,), jnp.int32)])
    emb_flat = _sc_gather(embedding_table, idx)

    yf_t = pv_y_osgb_fourier.transpose(0, 2, 1)
    xf_t = pv_x_osgb_fourier.transpose(0, 2, 1)
    emb_t = emb_flat[:_E * _N].reshape(_E, _N, _DE).transpose(0, 2, 1)
    sal = solar_azimuth.reshape(_E, _T, 1)
    sel = solar_elevation.reshape(_E, _T, 1)
    pad_t = query_padding.T

    out = pl.pallas_call(
        _assemble_body,
        grid=(_E // _EB,),
        in_specs=[
            pl.BlockSpec((_EB, _P, _N), lambda g: (g, 0, 0)),
            pl.BlockSpec((_EB, _P, _N), lambda g: (g, 0, 0)),
            pl.BlockSpec((_EB, _DE, _N), lambda g: (g, 0, 0)),
            pl.BlockSpec((_EB, _T, 1), lambda g: (g, 0, 0)),
            pl.BlockSpec((_EB, _T, 1), lambda g: (g, 0, 0)),
            pl.BlockSpec((_QD, _P), lambda g: (0, 0)),
        ],
        out_specs=pl.BlockSpec((_QD, _EB * _T, _P + _N),
                               lambda g: (0, g, 0)),
        out_shape=jax.ShapeDtypeStruct((_QD, _E * _T, _P + _N), jnp.float32),
        compiler_params=pltpu.CompilerParams(
            dimension_semantics=("parallel",)),
    )(yf_t, xf_t, emb_t, sal, sel, pad_t)
    return jnp.transpose(out, (1, 2, 0))


# speedup vs baseline: 1.2447x; 1.1248x over previous
import functools

import jax
import jax.numpy as jnp
from jax import lax
from jax.experimental import pallas as pl
from jax.experimental.pallas import tpu as pltpu
from jax.experimental.pallas import tpu_sc as plsc

_E, _T, _N = 16, 32, 1400
_P = 8
_DE = 16
_QD = 34
_NC, _NS = 2, 16
_NW = _NC * _NS
_BPW = 704
_BPAD = _NW * _BPW


def _sc_gather(table, idx_flat):
    mesh = plsc.VectorSubcoreMesh(core_axis_name="c", subcore_axis_name="s")

    n_full = _E * _N - (_NW - 1) * _BPW

    @functools.partial(
        pl.kernel,
        mesh=mesh,
        out_type=jax.ShapeDtypeStruct((_E * _N, _DE), jnp.float32),
        scratch_types=[
            pltpu.VMEM((_BPW,), jnp.int32),
            pltpu.VMEM((_BPW, _DE), jnp.float32),
            pltpu.SemaphoreType.DMA,
        ],
        compiler_params=pltpu.CompilerParams(use_tc_tiling_on_sc=False),
    )
    def gather_kernel(table_hbm, idx_hbm, out_hbm, idx_v, rows_v, sem):
        wid = lax.axis_index("s") * _NC + lax.axis_index("c")
        base = wid * _BPW
        pltpu.sync_copy(idx_hbm.at[pl.ds(base, _BPW)], idx_v)
        pltpu.async_copy(table_hbm.at[idx_v], rows_v, sem).wait()

        @pl.when(base + _BPW <= _E * _N)
        def _():
            pltpu.sync_copy(rows_v, out_hbm.at[pl.ds(base, _BPW)])

        @pl.when(base + _BPW > _E * _N)
        def _():
            pltpu.sync_copy(rows_v.at[pl.ds(0, n_full)],
                            out_hbm.at[pl.ds(base, n_full)])

    return gather_kernel(table, idx_flat)


_EB = 2


def _assemble_body(yf_ref, xf_ref, emb_ref, sal_ref, sel_ref, pad_ref,
                   out_ref):
    for c in range(_QD):
        for ex in range(_EB):
            if c < 16:
                ref = yf_ref if c < 8 else xf_ref
                src = jnp.nan_to_num(ref[ex, c % 8, :].reshape(1, _N))
                data = jnp.broadcast_to(src, (_T, _N))
            elif c < 32:
                src = jnp.nan_to_num(emb_ref[ex, c - 16, :].reshape(1, _N))
                data = jnp.broadcast_to(src, (_T, _N))
            else:
                ref = sal_ref if c == 32 else sel_ref
                src = jnp.nan_to_num(ref[ex, :, :])
                data = jnp.broadcast_to(src, (_T, _N))
            padv = jnp.broadcast_to(pad_ref[c, :].reshape(1, _P), (_T, _P))
            out_ref[c, ex * _T:(ex + 1) * _T, :] = jnp.concatenate(
                [padv, data], axis=1)


def kernel(pv, pv_y_osgb_fourier, pv_x_osgb_fourier, pv_system_row_number,
           pv_x_osgb, solar_azimuth, solar_elevation, query_padding,
           embedding_table):
    idx = pv_system_row_number.reshape(-1)
    idx = jnp.concatenate(
        [idx, jnp.zeros((_BPAD - _E * _N,), jnp.int32)])
    emb_flat = _sc_gather(embedding_table, idx)

    yf_t = pv_y_osgb_fourier.transpose(0, 2, 1)
    xf_t = pv_x_osgb_fourier.transpose(0, 2, 1)
    emb_t = emb_flat.reshape(_E, _N, _DE).transpose(0, 2, 1)
    sal = solar_azimuth.reshape(_E, _T, 1)
    sel = solar_elevation.reshape(_E, _T, 1)
    pad_t = query_padding.T

    out = pl.pallas_call(
        _assemble_body,
        grid=(_E // _EB,),
        in_specs=[
            pl.BlockSpec((_EB, _P, _N), lambda g: (g, 0, 0)),
            pl.BlockSpec((_EB, _P, _N), lambda g: (g, 0, 0)),
            pl.BlockSpec((_EB, _DE, _N), lambda g: (g, 0, 0)),
            pl.BlockSpec((_EB, _T, 1), lambda g: (g, 0, 0)),
            pl.BlockSpec((_EB, _T, 1), lambda g: (g, 0, 0)),
            pl.BlockSpec((_QD, _P), lambda g: (0, 0)),
        ],
        out_specs=pl.BlockSpec((_QD, _EB * _T, _P + _N),
                               lambda g: (0, g, 0)),
        out_shape=jax.ShapeDtypeStruct((_QD, _E * _T, _P + _N), jnp.float32),
        compiler_params=pltpu.CompilerParams(
            dimension_semantics=("parallel",)),
    )(yf_t, xf_t, emb_t, sal, sel, pad_t)
    return jnp.transpose(out, (1, 2, 0))


# speedup vs baseline: 1.3511x; 1.0855x over previous
import functools

import jax
import jax.numpy as jnp
from jax import lax
from jax.experimental import pallas as pl
from jax.experimental.pallas import tpu as pltpu
from jax.experimental.pallas import tpu_sc as plsc

_E, _T, _N = 16, 32, 1400
_P = 8
_DE = 16
_QD = 34
_NC, _NS = 2, 16
_NW = _NC * _NS
_BPW = 704
_BPAD = _NW * _BPW


def _sc_gather(table, idx_flat):
    mesh = plsc.VectorSubcoreMesh(core_axis_name="c", subcore_axis_name="s")

    n_full = _E * _N - (_NW - 1) * _BPW

    @functools.partial(
        pl.kernel,
        mesh=mesh,
        out_type=jax.ShapeDtypeStruct((_E * _N, _DE), jnp.float32),
        scratch_types=[
            pltpu.VMEM((_BPW,), jnp.int32),
            pltpu.VMEM((_BPW, _DE), jnp.float32),
            pltpu.SemaphoreType.DMA,
        ],
        compiler_params=pltpu.CompilerParams(use_tc_tiling_on_sc=False),
    )
    def gather_kernel(table_hbm, idx_hbm, out_hbm, idx_v, rows_v, sem):
        wid = lax.axis_index("s") * _NC + lax.axis_index("c")
        base = wid * _BPW
        pltpu.sync_copy(idx_hbm.at[pl.ds(base, _BPW)], idx_v)
        pltpu.async_copy(table_hbm.at[idx_v], rows_v, sem).wait()

        @pl.when(base + _BPW <= _E * _N)
        def _():
            pltpu.sync_copy(rows_v, out_hbm.at[pl.ds(base, _BPW)])

        @pl.when(base + _BPW > _E * _N)
        def _():
            pltpu.sync_copy(rows_v.at[pl.ds(0, n_full)],
                            out_hbm.at[pl.ds(base, n_full)])

    return gather_kernel(table, idx_flat)


_EB = 2


def _assemble_body(yf_ref, xf_ref, emb_ref, sa_ref, se_ref, pad_ref,
                   out_ref):
    g = pl.program_id(0)
    sub = lax.broadcasted_iota(jnp.int32, (_T, 1), 0)
    for ex in range(_EB):
        e = g * _EB + ex
        emb_t = jnp.transpose(emb_ref[ex * _N:(ex + 1) * _N, :])
        sa = jnp.zeros((_T, 1), jnp.float32)
        se = jnp.zeros((_T, 1), jnp.float32)
        for t in range(_T):
            sa = jnp.where(sub == t, sa_ref[e, t], sa)
            se = jnp.where(sub == t, se_ref[e, t], se)
        sa = jnp.nan_to_num(sa)
        se = jnp.nan_to_num(se)
        for c in range(_QD):
            if c < 16:
                ref = yf_ref if c < 8 else xf_ref
                src = jnp.nan_to_num(ref[ex, c % 8, :].reshape(1, _N))
                data = jnp.broadcast_to(src, (_T, _N))
            elif c < 32:
                src = jnp.nan_to_num(emb_t[c - 16, :].reshape(1, _N))
                data = jnp.broadcast_to(src, (_T, _N))
            else:
                data = jnp.broadcast_to(sa if c == 32 else se, (_T, _N))
            padv = jnp.broadcast_to(pad_ref[c, :].reshape(1, _P), (_T, _P))
            out_ref[c, ex * _T:(ex + 1) * _T, :] = jnp.concatenate(
                [padv, data], axis=1)


def kernel(pv, pv_y_osgb_fourier, pv_x_osgb_fourier, pv_system_row_number,
           pv_x_osgb, solar_azimuth, solar_elevation, query_padding,
           embedding_table):
    idx = pv_system_row_number.reshape(-1)
    idx = jnp.concatenate(
        [idx, jnp.zeros((_BPAD - _E * _N,), jnp.int32)])
    emb_flat = _sc_gather(embedding_table, idx)

    yf_t = pv_y_osgb_fourier.transpose(0, 2, 1)
    xf_t = pv_x_osgb_fourier.transpose(0, 2, 1)
    pad_t = query_padding.T

    out = pl.pallas_call(
        _assemble_body,
        grid=(_E // _EB,),
        in_specs=[
            pl.BlockSpec((_EB, _P, _N), lambda g: (g, 0, 0)),
            pl.BlockSpec((_EB, _P, _N), lambda g: (g, 0, 0)),
            pl.BlockSpec((_EB * _N, _DE), lambda g: (g, 0)),
            pl.BlockSpec(memory_space=pltpu.SMEM),
            pl.BlockSpec(memory_space=pltpu.SMEM),
            pl.BlockSpec((_QD, _P), lambda g: (0, 0)),
        ],
        out_specs=pl.BlockSpec((_QD, _EB * _T, _P + _N),
                               lambda g: (0, g, 0)),
        out_shape=jax.ShapeDtypeStruct((_QD, _E * _T, _P + _N), jnp.float32),
        compiler_params=pltpu.CompilerParams(
            dimension_semantics=("parallel",)),
    )(yf_t, xf_t, emb_flat, solar_azimuth, solar_elevation, pad_t)
    return jnp.transpose(out, (1, 2, 0))
